# baseline (speedup 1.0000x reference)
import functools

import jax
import jax.numpy as jnp
from jax import lax
from jax.experimental import pallas as pl
from jax.experimental.pallas import tpu as pltpu
from jax.experimental.pallas import tpu_sc as plsc

DA = 144
CH = 128
NROWS_PAD = 10240


def _sc_aggregate(xa, colp, rowp, n_chunks_per_worker):
    info = plsc.get_sparse_core_info()
    nc, ns = info.num_cores, info.num_subcores
    rows_per_tile = NROWS_PAD // ns
    mesh = plsc.VectorSubcoreMesh(core_axis_name="c", subcore_axis_name="s")
    kpw = n_chunks_per_worker

    @functools.partial(
        pl.kernel,
        mesh=mesh,
        compiler_params=pltpu.CompilerParams(use_tc_tiling_on_sc=False),
        out_type=jax.ShapeDtypeStruct((nc, NROWS_PAD, DA), jnp.float32),
        scratch_types=[
            pltpu.VMEM((CH,), jnp.int32),
            pltpu.VMEM((CH,), jnp.int32),
            pltpu.VMEM((CH,), jnp.int32),
            pltpu.VMEM((CH,), jnp.int32),
            pltpu.VMEM((CH, DA), jnp.float32),
            pltpu.VMEM((CH, DA), jnp.float32),
            pltpu.VMEM_SHARED((NROWS_PAD, DA), jnp.float32),
            pltpu.SemaphoreType.DMA,
            pltpu.SemaphoreType.DMA,
            pltpu.SemaphoreType.DMA,
            pltpu.SemaphoreType.DMA,
        ],
    )
    def k(xa_hbm, col_hbm, row_hbm, out_hbm, colc0, colc1, rowc0, rowc1,
          rows0, rows1, agg, gsem0, gsem1, isem0, isem1):
        c = lax.axis_index("c")
        s = lax.axis_index("s")
        wid = c * ns + s
        colc = (colc0, colc1)
        rowc = (rowc0, rowc1)
        rows = (rows0, rows1)
        gsem = (gsem0, gsem1)
        isem = (isem0, isem1)

        def zrow(i, carry):
            for j in range(DA // 16):
                rows0[i, pl.ds(j * 16, 16)] = jnp.zeros((16,), jnp.float32)
            return carry

        lax.fori_loop(0, CH, zrow, 0)
        for t in range(rows_per_tile // CH):
            pltpu.sync_copy(rows0, agg.at[pl.ds(s * rows_per_tile + t * CH, CH)])
        plsc.subcore_barrier()

        base_e = wid * kpw * CH

        def idx_fetch(g, buf):
            e0 = base_e + g * CH
            pltpu.async_copy(col_hbm.at[pl.ds(e0, CH)], colc[buf], isem[buf])
            pltpu.async_copy(row_hbm.at[pl.ds(e0, CH)], rowc[buf], isem[buf])

        def idx_wait(buf):
            e0 = base_e
            pltpu.make_async_copy(
                col_hbm.at[pl.ds(e0, CH)], colc[buf], isem[buf]).wait()
            pltpu.make_async_copy(
                row_hbm.at[pl.ds(e0, CH)], rowc[buf], isem[buf]).wait()

        def gather_launch(buf):
            pltpu.async_copy(xa_hbm.at[colc[buf]], rows[buf], gsem[buf])

        def gather_wait(buf):
            pltpu.make_async_copy(
                xa_hbm.at[colc[buf]], rows[buf], gsem[buf]).wait()

        idx_fetch(0, 0)
        idx_fetch(1, 1)
        idx_wait(0)
        gather_launch(0)

        def body(h, carry):
            for p in range(2):
                g = 2 * h + p

                @pl.when(g + 1 < kpw)
                def _():
                    idx_wait(1 - p)
                    gather_launch(1 - p)

                gather_wait(p)
                pltpu.sync_copy(rows[p], agg.at[rowc[p]], add=True)

                @pl.when(g + 2 < kpw)
                def _():
                    idx_fetch(g + 2, p)
            return carry

        lax.fori_loop(0, kpw // 2, body, 0)
        plsc.subcore_barrier()
        pltpu.sync_copy(
            agg.at[pl.ds(s * rows_per_tile, rows_per_tile)],
            out_hbm.at[c, pl.ds(s * rows_per_tile, rows_per_tile)],
        )

    return k(xa, colp, rowp)


def _tc_finish_body(xa_ref, p0_ref, p1_ref, wa_ref, g_ref, b_ref, o_ref):
    s = xa_ref[...] + p0_ref[...] + p1_ref[...]
    h = lax.dot_general(
        s, wa_ref[...], (((1,), (1,)), ((), ())),
        preferred_element_type=jnp.float32,
    )
    mean = jnp.mean(h, axis=1, keepdims=True)
    d = h - mean
    var = jnp.mean(d * d, axis=1, keepdims=True)
    y = d * lax.rsqrt(var + 1e-5) * g_ref[...] + b_ref[...]
    o_ref[...] = jnp.maximum(y, 0.0)


def _tc_finish(xa, p0, p1, wa, gamma2, beta2):
    n = xa.shape[0]
    dout = wa.shape[0]
    bs = 2000
    grid = n // bs
    return pl.pallas_call(
        _tc_finish_body,
        grid=(grid,),
        in_specs=[
            pl.BlockSpec((bs, DA), lambda i: (i, 0)),
            pl.BlockSpec((bs, DA), lambda i: (i, 0)),
            pl.BlockSpec((bs, DA), lambda i: (i, 0)),
            pl.BlockSpec((dout, DA), lambda i: (0, 0)),
            pl.BlockSpec((1, dout), lambda i: (0, 0)),
            pl.BlockSpec((1, dout), lambda i: (0, 0)),
        ],
        out_specs=pl.BlockSpec((bs, dout), lambda i: (i, 0)),
        out_shape=jax.ShapeDtypeStruct((n, dout), jnp.float32),
    )(xa, p0, p1, wa, gamma2, beta2)


def kernel(x, edge_index, W, b, gamma, beta):
    n, d_in = x.shape
    d_out = W.shape[0]
    e = edge_index.shape[1]
    row = edge_index[0].astype(jnp.int32)
    col = edge_index[1].astype(jnp.int32)

    xa = jnp.concatenate(
        [x, jnp.ones((n, 1), jnp.float32), jnp.zeros((n, DA - d_in - 1), jnp.float32)],
        axis=1,
    )
    wa = jnp.concatenate(
        [W, b[:, None], jnp.zeros((d_out, DA - d_in - 1), jnp.float32)], axis=1
    )

    info = plsc.get_sparse_core_info()
    nw = info.num_cores * info.num_subcores
    kpw = -(-e // (nw * CH))
    kpw += kpw % 2
    e_pad = kpw * nw * CH
    trash = n
    colp = jnp.concatenate([col, jnp.zeros((e_pad - e,), jnp.int32)])
    rowp = jnp.concatenate([row, jnp.full((e_pad - e,), trash, jnp.int32)])

    parts = _sc_aggregate(xa, colp, rowp, kpw)
    p0 = parts[0, :n]
    p1 = parts[1, :n]
    return _tc_finish(xa, p0, p1, wa, gamma.reshape(1, d_out), beta.reshape(1, d_out))


# speedup vs baseline: 1.1129x; 1.1129x over previous
import functools

import jax
import jax.numpy as jnp
from jax import lax
from jax.experimental import pallas as pl
from jax.experimental.pallas import tpu as pltpu
from jax.experimental.pallas import tpu_sc as plsc

DA = 144
CH = 128
NROWS_PAD = 10240


def _sc_aggregate(xa, colp, rowp, kpw0, kpw1):
    info = plsc.get_sparse_core_info()
    nc, ns = info.num_cores, info.num_subcores
    rows_per_tile = NROWS_PAD // ns
    mesh = plsc.VectorSubcoreMesh(core_axis_name="c", subcore_axis_name="s")

    @functools.partial(
        pl.kernel,
        mesh=mesh,
        compiler_params=pltpu.CompilerParams(use_tc_tiling_on_sc=False),
        out_type=jax.ShapeDtypeStruct((nc, NROWS_PAD, DA), jnp.float32),
        scratch_types=(
            [pltpu.VMEM((CH,), jnp.int32)] * 4
            + [pltpu.VMEM((CH,), jnp.int32)] * 4
            + [pltpu.VMEM((CH, DA), jnp.float32)] * 2
            + [pltpu.VMEM_SHARED((NROWS_PAD, DA), jnp.float32)]
            + [pltpu.SemaphoreType.DMA] * 8
        ),
    )
    def k(xa_hbm, col_hbm, row_hbm, out_hbm,
          cc0, cc1, cc2, cc3, rc0, rc1, rc2, rc3, rows0, rows1, agg,
          is0, is1, is2, is3, gsem0, gsem1, ssem0, ssem1):
        c = lax.axis_index("c")
        s = lax.axis_index("s")
        kpw = jnp.where(c == 0, kpw0, kpw1)
        base_chunk = c * ns * kpw0 + s * kpw
        colc = (cc0, cc1, cc2, cc3)
        rowc = (rc0, rc1, rc2, rc3)
        rows = (rows0, rows1)
        isem = (is0, is1, is2, is3)
        gsem = (gsem0, gsem1)
        ssem = (ssem0, ssem1)

        def zrow(i, carry):
            for j in range(DA // 16):
                rows0[i, pl.ds(j * 16, 16)] = jnp.zeros((16,), jnp.float32)
            return carry

        lax.fori_loop(0, CH, zrow, 0)
        for t in range(rows_per_tile // CH):
            pltpu.sync_copy(rows0, agg.at[pl.ds(s * rows_per_tile + t * CH, CH)])
        plsc.subcore_barrier()

        base_e = base_chunk * CH

        def idx_fetch(g, i):
            e0 = base_e + g * CH
            pltpu.async_copy(col_hbm.at[pl.ds(e0, CH)], colc[i], isem[i])
            pltpu.async_copy(row_hbm.at[pl.ds(e0, CH)], rowc[i], isem[i])

        def idx_wait(i):
            pltpu.make_async_copy(
                col_hbm.at[pl.ds(base_e, CH)], colc[i], isem[i]).wait()
            pltpu.make_async_copy(
                row_hbm.at[pl.ds(base_e, CH)], rowc[i], isem[i]).wait()

        def gather_launch(p, i):
            pltpu.async_copy(xa_hbm.at[colc[i]], rows[p], gsem[p])

        def gather_wait(p, i):
            pltpu.make_async_copy(
                xa_hbm.at[colc[i]], rows[p], gsem[p]).wait()

        def scatter_launch(p, i):
            pltpu.async_copy(rows[p], agg.at[rowc[i]], ssem[p], add=True)

        def scatter_wait(p, i):
            pltpu.make_async_copy(
                rows[p], agg.at[rowc[i]], ssem[p]).wait()

        idx_fetch(0, 0)
        idx_fetch(1, 1)
        idx_wait(0)
        gather_launch(0, 0)

        def body(h, carry):
            for q in range(4):
                g = 4 * h + q
                p = q % 2
                i = q

                @pl.when((g + 1 < kpw) & (g >= 1))
                def _():
                    scatter_wait(1 - p, (i + 3) % 4)

                @pl.when(g + 1 < kpw)
                def _():
                    idx_wait((i + 1) % 4)
                    gather_launch(1 - p, (i + 1) % 4)

                gather_wait(p, i)
                scatter_launch(p, i)

                @pl.when(g + 2 < kpw)
                def _():
                    idx_fetch(g + 2, (i + 2) % 4)
            return carry

        lax.fori_loop(0, kpw // 4, body, 0)
        scatter_wait(0, 0)
        scatter_wait(1, 0)
        plsc.subcore_barrier()
        pltpu.sync_copy(
            agg.at[pl.ds(s * rows_per_tile, rows_per_tile)],
            out_hbm.at[c, pl.ds(s * rows_per_tile, rows_per_tile)],
        )

    return k(xa, colp, rowp)


def _tc_finish_body(xa_ref, p0_ref, p1_ref, wa_ref, g_ref, b_ref, o_ref):
    s = xa_ref[...] + p0_ref[...] + p1_ref[...]
    h = lax.dot_general(
        s, wa_ref[...], (((1,), (1,)), ((), ())),
        preferred_element_type=jnp.float32,
    )
    mean = jnp.mean(h, axis=1, keepdims=True)
    d = h - mean
    var = jnp.mean(d * d, axis=1, keepdims=True)
    y = d * lax.rsqrt(var + 1e-5) * g_ref[...] + b_ref[...]
    o_ref[...] = jnp.maximum(y, 0.0)


def _tc_finish(xa, p0, p1, wa, gamma2, beta2):
    n = xa.shape[0]
    dout = wa.shape[0]
    bs = 2000
    grid = n // bs
    return pl.pallas_call(
        _tc_finish_body,
        grid=(grid,),
        in_specs=[
            pl.BlockSpec((bs, DA), lambda i: (i, 0)),
            pl.BlockSpec((bs, DA), lambda i: (i, 0)),
            pl.BlockSpec((bs, DA), lambda i: (i, 0)),
            pl.BlockSpec((dout, DA), lambda i: (0, 0)),
            pl.BlockSpec((1, dout), lambda i: (0, 0)),
            pl.BlockSpec((1, dout), lambda i: (0, 0)),
        ],
        out_specs=pl.BlockSpec((bs, dout), lambda i: (i, 0)),
        out_shape=jax.ShapeDtypeStruct((n, dout), jnp.float32),
    )(xa, p0, p1, wa, gamma2, beta2)


def kernel(x, edge_index, W, b, gamma, beta):
    n, d_in = x.shape
    d_out = W.shape[0]
    e = edge_index.shape[1]
    row = edge_index[0].astype(jnp.int32)
    col = edge_index[1].astype(jnp.int32)

    xa = jnp.concatenate(
        [x, jnp.ones((n, 1), jnp.float32), jnp.zeros((n, DA - d_in - 1), jnp.float32)],
        axis=1,
    )
    wa = jnp.concatenate(
        [W, b[:, None], jnp.zeros((d_out, DA - d_in - 1), jnp.float32)], axis=1
    )

    info = plsc.get_sparse_core_info()
    ns = info.num_subcores
    tot = -(-e // CH)
    frac0 = 0.775
    kpw0 = (int(tot * frac0) // ns + 4) // 4 * 4
    rem = max(0, tot - ns * kpw0)
    kpw1 = max(4, (-(-rem // ns) + 3) // 4 * 4)
    e_pad = ns * (kpw0 + kpw1) * CH
    trash = n
    colp = jnp.concatenate([col, jnp.zeros((e_pad - e,), jnp.int32)])
    rowp = jnp.concatenate([row, jnp.full((e_pad - e,), trash, jnp.int32)])

    parts = _sc_aggregate(xa, colp, rowp, kpw0, kpw1)
    p0 = parts[0, :n]
    p1 = parts[1, :n]
    return _tc_finish(xa, p0, p1, wa, gamma.reshape(1, d_out), beta.reshape(1, d_out))
